# baseline (speedup 1.0000x reference)
import functools

import numpy as np
import jax
from jax import lax
import jax.numpy as jnp
from jax.experimental import pallas as pl
from jax.experimental.pallas import tpu as pltpu
from jax.experimental.pallas import tpu_sc as plsc


@functools.lru_cache
def _masked_positions(S):
    with jax.ensure_compile_time_eval():
        idx_arr = jax.random.choice(
            jax.random.key(42), S, shape=(3,), replace=False)
        return tuple(sorted(int(v) for v in np.asarray(idx_arr)))


def _tc_copy_body(nblocks, blk, W, x_ref, o_ref):
    def inner(x_blk, o_blk):
        o_blk[...] = x_blk[...]

    pltpu.emit_pipeline(
        inner,
        grid=(nblocks,),
        in_specs=[pl.BlockSpec((blk, W), lambda i: (i, 0))],
        out_specs=[pl.BlockSpec((blk, W), lambda i: (i, 0))],
    )(x_ref, o_ref)


def _sc_scatter_body(mask_rows, E, nc, o_ref, mt_ref, mtbuf, msem):
    wid = lax.axis_index("s") * nc + lax.axis_index("c")
    for k, row in enumerate(mask_rows):
        @pl.when(wid == k)
        def _():
            pltpu.async_copy(
                mt_ref, o_ref.at[pl.ds(row, 1), :], msem).wait()


def kernel(x, mask_token):
    B, S, E = x.shape
    idx = _masked_positions(S)

    R = B * S
    mask_rows = [b * S + s for b in range(B) for s in idx]

    f32 = jnp.float32
    xf = x.reshape(R, E)
    mtf = mask_token.reshape(1, E)
    oref = jax.empty_ref(jax.ShapeDtypeStruct((R, E), f32))

    blk = 2048
    nblocks = R // blk
    tc_copy = pl.kernel(
        functools.partial(_tc_copy_body, nblocks, blk, E),
        mesh=pltpu.create_tensorcore_mesh("t"),
        out_type=(),
    )
    tc_copy(xf, oref)

    info = plsc.get_sparse_core_info()
    nc = info.num_cores
    sc_scatter = pl.kernel(
        functools.partial(_sc_scatter_body, mask_rows, E, nc),
        mesh=plsc.VectorSubcoreMesh(core_axis_name="c", subcore_axis_name="s"),
        out_type=(),
        scratch_types=[
            pltpu.VMEM((1, E), f32),
            pltpu.SemaphoreType.DMA,
        ],
        compiler_params=pltpu.CompilerParams(use_tc_tiling_on_sc=True),
    )
    sc_scatter(oref, mtf)
    return jax.freeze(oref).reshape(B, S, E)


# speedup vs baseline: 1.0110x; 1.0110x over previous
import functools

import numpy as np
import jax
from jax import lax
import jax.numpy as jnp
from jax.experimental import pallas as pl
from jax.experimental.pallas import tpu as pltpu
from jax.experimental.pallas import tpu_sc as plsc


@functools.lru_cache
def _masked_positions(S):
    with jax.ensure_compile_time_eval():
        idx_arr = jax.random.choice(
            jax.random.key(42), S, shape=(3,), replace=False)
        return tuple(sorted(int(v) for v in np.asarray(idx_arr)))


def _tc_copy_body(nblocks, blk, W, x_ref, o_ref):
    def inner(x_blk, o_blk):
        o_blk[...] = x_blk[...]

    pltpu.emit_pipeline(
        inner,
        grid=(nblocks,),
        in_specs=[pl.BlockSpec((blk, W), lambda i: (i, 0))],
        out_specs=[pl.BlockSpec((blk, W), lambda i: (i, 0))],
    )(x_ref, o_ref)


def _sc_scatter_body(mask_rows, E, nc, nw, o_ref, mt_ref, mtbuf, msem):
    wid = lax.axis_index("s") * nc + lax.axis_index("c")
    for k, row in enumerate(mask_rows):
        @pl.when(wid == k % nw)
        def _():
            pltpu.async_copy(mt_ref, mtbuf, msem).wait()
            pltpu.async_copy(
                mtbuf, o_ref.at[pl.ds(row, 1), :], msem).wait()


def kernel(x, mask_token):
    B, S, E = x.shape
    idx = _masked_positions(S)

    R = B * S
    mask_rows = [b * S + s for b in range(B) for s in idx]

    f32 = jnp.float32
    xf = x.reshape(R, E)
    mtf = mask_token.reshape(1, E)
    oref = jax.empty_ref(jax.ShapeDtypeStruct((R, E), f32))

    blk = 2048
    while R % blk:
        blk //= 2
    nblocks = R // blk
    tc_copy = pl.kernel(
        functools.partial(_tc_copy_body, nblocks, blk, E),
        mesh=pltpu.create_tensorcore_mesh("t"),
        out_type=(),
    )
    tc_copy(xf, oref)

    info = plsc.get_sparse_core_info()
    nc = info.num_cores
    nw = nc * info.num_subcores
    sc_scatter = pl.kernel(
        functools.partial(_sc_scatter_body, mask_rows, E, nc, nw),
        mesh=plsc.VectorSubcoreMesh(core_axis_name="c", subcore_axis_name="s"),
        out_type=(),
        scratch_types=[
            pltpu.VMEM((1, E), f32),
            pltpu.SemaphoreType.DMA,
        ],
        compiler_params=pltpu.CompilerParams(use_tc_tiling_on_sc=True),
    )
    sc_scatter(oref, mtf)
    return jax.freeze(oref).reshape(B, S, E)
